# baseline (speedup 1.0000x reference)
import jax
import jax.numpy as jnp
from jax import lax
from jax.experimental import pallas as pl
from jax.experimental.pallas import tpu as pltpu
from jax.experimental.pallas import tpu_sc as plsc

BATCH = 4096
HIST = 200
WORD_DIM = 64
AGE_DIM = 32
OUT_DIM = WORD_DIM + AGE_DIM

NTOK = BATCH * HIST
IDXW = 128
NROWS = NTOK // IDXW
NWORKERS = 32
ROWS_PER_W = NROWS // NWORKERS
RPC = 4
CHUNK = RPC * IDXW
NCHUNKS = ROWS_PER_W // RPC


def _body(widx_hbm, aidx_hbm, wtab_hbm, atab_hbm, out_hbm,
          widx_v, aidx_v, wrows_v, arows_v, isem, gsem0, gsem1, wsem):
    cid = lax.axis_index("c")
    sid = lax.axis_index("s")
    wid = sid * 2 + cid
    row_base = wid * ROWS_PER_W
    gsems = (gsem0, gsem1)

    def start_idx(i, slot):
        r = row_base + i * RPC
        pltpu.async_copy(widx_hbm.at[pl.ds(r, RPC)], widx_v.at[slot], isem)
        pltpu.async_copy(aidx_hbm.at[pl.ds(r, RPC)], aidx_v.at[slot], isem)

    def wait_idx(slot):
        pltpu.make_async_copy(
            widx_hbm.at[pl.ds(0, RPC)], widx_v.at[slot], isem).wait()
        pltpu.make_async_copy(
            aidx_hbm.at[pl.ds(0, RPC)], aidx_v.at[slot], isem).wait()

    def fire_gathers(slot):
        for j in range(RPC):
            pltpu.async_copy(wtab_hbm.at[widx_v.at[slot, j]],
                             wrows_v.at[slot, pl.ds(j * IDXW, IDXW)], gsems[slot])
            pltpu.async_copy(atab_hbm.at[aidx_v.at[slot, j]],
                             arows_v.at[slot, pl.ds(j * IDXW, IDXW)], gsems[slot])

    def wait_gathers(slot):
        pltpu.make_async_copy(
            wtab_hbm.at[pl.ds(0, CHUNK)], wrows_v.at[slot], gsems[slot]).wait()
        pltpu.make_async_copy(
            out_hbm.at[pl.ds(0, CHUNK), pl.ds(WORD_DIM, AGE_DIM)],
            arows_v.at[slot], gsems[slot]).wait()

    def start_writes(i, slot):
        base = (row_base + i * RPC) * IDXW
        pltpu.async_copy(
            wrows_v.at[slot],
            out_hbm.at[pl.ds(base, CHUNK), pl.ds(0, WORD_DIM)], wsem)
        pltpu.async_copy(
            arows_v.at[slot],
            out_hbm.at[pl.ds(base, CHUNK), pl.ds(WORD_DIM, AGE_DIM)], wsem)

    def wait_writes(slot):
        pltpu.make_async_copy(
            wrows_v.at[slot],
            out_hbm.at[pl.ds(0, CHUNK), pl.ds(0, WORD_DIM)], wsem).wait()
        pltpu.make_async_copy(
            arows_v.at[slot],
            out_hbm.at[pl.ds(0, CHUNK), pl.ds(WORD_DIM, AGE_DIM)], wsem).wait()

    start_idx(0, 0)
    wait_idx(0)
    fire_gathers(0)
    start_idx(1, 1)
    wait_idx(1)
    fire_gathers(1)
    wait_gathers(0)
    start_idx(2, 0)
    start_writes(0, 0)
    wait_idx(0)

    def step2(t, carry):
        for k in range(2):
            i = 2 + 2 * t + k
            slot, prev = k, 1 - k
            wait_writes(slot)
            fire_gathers(slot)
            wait_gathers(prev)
            start_idx(jnp.minimum(i + 1, NCHUNKS - 1), prev)
            start_writes(i - 1, prev)
            wait_idx(prev)
        return carry

    lax.fori_loop(0, (NCHUNKS - 2) // 2, step2, 0)

    last_slot = (NCHUNKS - 1) % 2
    wait_gathers(last_slot)
    start_writes(NCHUNKS - 1, last_slot)
    wait_writes(0)
    wait_writes(1)


@jax.jit
def _embed(widx, aidx, word_table, age_table):
    kern = pl.kernel(
        _body,
        out_type=jax.ShapeDtypeStruct((NTOK, OUT_DIM), jnp.float32),
        mesh=plsc.VectorSubcoreMesh(core_axis_name="c", subcore_axis_name="s"),
        scratch_types=[
            pltpu.VMEM((2, RPC, IDXW), jnp.int32),
            pltpu.VMEM((2, RPC, IDXW), jnp.int32),
            pltpu.VMEM((2, CHUNK, WORD_DIM), jnp.float32),
            pltpu.VMEM((2, CHUNK, AGE_DIM), jnp.float32),
            pltpu.SemaphoreType.DMA,
            pltpu.SemaphoreType.DMA,
            pltpu.SemaphoreType.DMA,
            pltpu.SemaphoreType.DMA,
        ],
        compiler_params=pltpu.CompilerParams(use_tc_tiling_on_sc=False),
    )
    return kern(widx, aidx, word_table, age_table)


def kernel(word, age, word_table, age_table):
    widx = word.astype(jnp.int32).reshape(NROWS, IDXW)
    aidx = age.astype(jnp.int32).reshape(NROWS, IDXW)
    out = _embed(widx, aidx, word_table, age_table)
    return out.reshape(BATCH, HIST, OUT_DIM)
